# Initial kernel scaffold; baseline (speedup 1.0000x reference)
import jax
import jax.numpy as jnp
from jax.experimental import pallas as pl


def kernel(x, lin_ln1_g, lin_ln1_b, lin_wq, lin_wk, lin_wv, lin_pk, lin_pv, lin_wo, lin_bo, lin_ln2_g, lin_ln2_b, lin_w1, lin_b1, lin_w2, lin_b2, conv_ln1_g, conv_ln1_b, conv_wq, conv_wk, conv_wv, conv_pk, conv_pv, conv_wo, conv_bo, conv_ln2_g, conv_ln2_b, conv_w1, conv_b1, conv_w2, conv_b2):
    raise NotImplementedError("write your pallas kernel here")


# speedup vs baseline: 1.7064x; 1.7064x over previous
import jax
import jax.numpy as jnp
from jax.experimental import pallas as pl
from jax.experimental.pallas import tpu as pltpu

_INTERPRET = False
_H = 8


def _ln(x, g, b, eps=1e-5):
    m = jnp.mean(x, -1, keepdims=True)
    v = jnp.mean((x - m) ** 2, -1, keepdims=True)
    return (x - m) * jax.lax.rsqrt(v + eps) * g + b


def _pick_tile(total, want):
    t = min(want, total)
    while total % t:
        t //= 2
    return t


def _cparams(sem, vmem_mb=50):
    return pltpu.CompilerParams(
        dimension_semantics=sem, vmem_limit_bytes=vmem_mb * 1024 * 1024)


def _qkv_body(x_ref, g_ref, b_ref, wq_ref, wk_ref, wv_ref, q_ref, k_ref, v_ref):
    xn = _ln(x_ref[...], g_ref[...], b_ref[...])
    q_ref[...] = jnp.dot(xn, wq_ref[...], preferred_element_type=jnp.float32)
    k_ref[...] = jnp.dot(xn, wk_ref[...], preferred_element_type=jnp.float32)
    v_ref[...] = jnp.dot(xn, wv_ref[...], preferred_element_type=jnp.float32)


def _qkv(xf, g, b, wq, wk, wv):
    BN, D = xf.shape
    TN = _pick_tile(BN, 256)
    grid = (BN // TN,)
    row = pl.BlockSpec((TN, D), lambda i: (i, 0))
    full = pl.BlockSpec((D, D), lambda i: (0, 0))
    vec = pl.BlockSpec((1, D), lambda i: (0, 0))
    out = jax.ShapeDtypeStruct((BN, D), jnp.float32)
    return pl.pallas_call(
        _qkv_body,
        grid=grid,
        in_specs=[row, vec, vec, full, full, full],
        out_specs=[row, row, row],
        out_shape=[out, out, out],
        compiler_params=_cparams(("parallel",)),
        name="ln_qkv",
        interpret=_INTERPRET,
    )(xf, g, b, wq, wk, wv)


def _linproj_body(pkT_ref, pvT_ref, ke_ref, va_ref, ko_ref, vo_ref):
    j = pl.program_id(1)
    Bn = ke_ref.shape[0]
    for bb in range(Bn):
        kk = jax.lax.dot_general(pkT_ref[...], ke_ref[bb], (((1,), (0,)), ((), ())),
                                 preferred_element_type=jnp.float32)
        vv = jax.lax.dot_general(pvT_ref[...], va_ref[bb], (((1,), (0,)), ((), ())),
                                 preferred_element_type=jnp.float32)

        @pl.when(j == 0)
        def _(bb=bb, kk=kk, vv=vv):
            ko_ref[bb] = kk
            vo_ref[bb] = vv

        @pl.when(j != 0)
        def _(bb=bb, kk=kk, vv=vv):
            ko_ref[bb] += kk
            vo_ref[bb] += vv


def _linproj(ke, va, pkT, pvT):
    B, N, D = ke.shape
    K = pkT.shape[0]
    BD = _pick_tile(D, D // 2)
    NC = _pick_tile(N, 1024)
    grid = (D // BD, N // NC)
    act = pl.BlockSpec((B, NC, BD), lambda d, j: (0, j, d))
    proj = pl.BlockSpec((K, NC), lambda d, j: (0, j))
    out = pl.BlockSpec((B, K, BD), lambda d, j: (0, 0, d))
    osh = jax.ShapeDtypeStruct((B, K, D), jnp.float32)
    return pl.pallas_call(
        _linproj_body,
        grid=grid,
        in_specs=[proj, proj, act, act],
        out_specs=[out, out],
        out_shape=[osh, osh],
        compiler_params=_cparams(("parallel", "arbitrary")),
        name="lin_kv_proj",
        interpret=_INTERPRET,
    )(pkT, pvT, ke, va)


def _convproj_body(pk_ref, pv_ref, ke_ref, va_ref, ko_ref, vo_ref):
    s = pl.program_id(1)
    Bn = ke_ref.shape[0]
    dn = (((1,), (0,)), ((), ()))
    for bb in range(Bn):
        kk = jax.lax.dot_general(ke_ref[bb], pk_ref[0], dn,
                                 preferred_element_type=jnp.float32)
        vv = jax.lax.dot_general(va_ref[bb], pv_ref[0], dn,
                                 preferred_element_type=jnp.float32)

        @pl.when(s == 0)
        def _(bb=bb, kk=kk, vv=vv):
            ko_ref[bb] = kk
            vo_ref[bb] = vv

        @pl.when(s != 0)
        def _(bb=bb, kk=kk, vv=vv):
            ko_ref[bb] += kk
            vo_ref[bb] += vv


def _convproj(ke5, va5, pkT, pvT):
    B, K, SD = ke5.shape
    S, C, D = pkT.shape
    BO = _pick_tile(D, D // 2)
    grid = (D // BO, S)
    act = pl.BlockSpec((B, K, C), lambda o, s: (0, 0, s))
    w = pl.BlockSpec((1, C, BO), lambda o, s: (s, 0, o))
    out = pl.BlockSpec((B, K, BO), lambda o, s: (0, 0, o))
    osh = jax.ShapeDtypeStruct((B, K, D), jnp.float32)
    return pl.pallas_call(
        _convproj_body,
        grid=grid,
        in_specs=[w, w, act, act],
        out_specs=[out, out],
        out_shape=[osh, osh],
        compiler_params=_cparams(("parallel", "arbitrary")),
        name="conv_kv_proj",
        interpret=_INTERPRET,
    )(pkT, pvT, ke5, va5)


def _attn_body(q_ref, k_ref, v_ref, x_ref, wo_ref, bo_ref, o_ref):
    q = q_ref[0]
    k = k_ref[0]
    v = v_ref[0]
    D = q.shape[1]
    DH = D // _H
    scale = DH ** -0.5
    outs = []
    for h in range(_H):
        sl = slice(h * DH, (h + 1) * DH)
        dots = jax.lax.dot_general(q[:, sl], k[:, sl], (((1,), (1,)), ((), ())),
                                   preferred_element_type=jnp.float32) * scale
        m = jnp.max(dots, axis=-1, keepdims=True)
        p = jnp.exp(dots - m)
        l = jnp.sum(p, axis=-1, keepdims=True)
        a = p / l
        outs.append(jax.lax.dot_general(a, v[:, sl], (((1,), (0,)), ((), ())),
                                        preferred_element_type=jnp.float32))
    o = jnp.concatenate(outs, axis=-1)
    o_ref[0] = x_ref[0] + jnp.dot(o, wo_ref[...], preferred_element_type=jnp.float32) + bo_ref[...]


def _attn(q3, k_, v_, x3, wo, bo):
    B, N, D = q3.shape
    K = k_.shape[1]
    TQ = _pick_tile(N, 256)
    grid = (B, N // TQ)
    row = pl.BlockSpec((1, TQ, D), lambda b, n: (b, n, 0))
    kv = pl.BlockSpec((1, K, D), lambda b, n: (b, 0, 0))
    full = pl.BlockSpec((D, D), lambda b, n: (0, 0))
    vec = pl.BlockSpec((1, D), lambda b, n: (0, 0))
    return pl.pallas_call(
        _attn_body,
        grid=grid,
        in_specs=[row, kv, kv, row, full, vec],
        out_specs=row,
        out_shape=jax.ShapeDtypeStruct((B, N, D), jnp.float32),
        compiler_params=_cparams(("parallel", "parallel")),
        name="attn_out",
        interpret=_INTERPRET,
    )(q3, k_, v_, x3, wo, bo)


def _ffn_body(x_ref, g_ref, b_ref, w1_ref, b1_ref, w2_ref, b2_ref, o_ref, *, nchunk):
    x = x_ref[...]
    xn = _ln(x, g_ref[...], b_ref[...])
    DFF = w1_ref.shape[1]
    CF = DFF // nchunk
    o_ref[...] = x + b2_ref[...]
    for c in range(nchunk):
        sl = slice(c * CF, (c + 1) * CF)
        h = jnp.dot(xn, w1_ref[:, sl], preferred_element_type=jnp.float32) + b1_ref[:, sl]
        h = 0.5 * h * (1.0 + jax.lax.erf(h * (2.0 ** -0.5)))
        o_ref[...] += jnp.dot(h, w2_ref[sl, :], preferred_element_type=jnp.float32)


def _ffn(xf, g, b, w1, b1, w2, b2):
    import functools
    BN, D = xf.shape
    DFF = w1.shape[1]
    TM = _pick_tile(BN, 256)
    CF = _pick_tile(DFF, 512)
    grid = (BN // TM,)
    row = pl.BlockSpec((TM, D), lambda i: (i, 0))
    return pl.pallas_call(
        functools.partial(_ffn_body, nchunk=DFF // CF),
        grid=grid,
        in_specs=[row,
                  pl.BlockSpec((1, D), lambda i: (0, 0)),
                  pl.BlockSpec((1, D), lambda i: (0, 0)),
                  pl.BlockSpec((D, DFF), lambda i: (0, 0)),
                  pl.BlockSpec((1, DFF), lambda i: (0, 0)),
                  pl.BlockSpec((DFF, D), lambda i: (0, 0)),
                  pl.BlockSpec((1, D), lambda i: (0, 0))],
        out_specs=row,
        out_shape=jax.ShapeDtypeStruct((BN, D), jnp.float32),
        compiler_params=_cparams(("parallel",)),
        name="ln_ffn",
        interpret=_INTERPRET,
    )(xf, g, b, w1, b1, w2, b2)


def _row(v):
    return v.reshape(1, -1)


def kernel(x, lin_ln1_g, lin_ln1_b, lin_wq, lin_wk, lin_wv, lin_pk, lin_pv,
           lin_wo, lin_bo, lin_ln2_g, lin_ln2_b, lin_w1, lin_b1, lin_w2, lin_b2,
           conv_ln1_g, conv_ln1_b, conv_wq, conv_wk, conv_wv, conv_pk, conv_pv,
           conv_wo, conv_bo, conv_ln2_g, conv_ln2_b, conv_w1, conv_b1, conv_w2, conv_b2):
    B, N, D = x.shape
    L = lin_wq.shape[0]
    K = lin_pk.shape[2]
    S = conv_pk.shape[3]
    xf = x.reshape(B * N, D)

    for i in range(L):
        q, ke, va = _qkv(xf, _row(lin_ln1_g[i]), _row(lin_ln1_b[i]),
                         lin_wq[i], lin_wk[i], lin_wv[i])
        k_, v_ = _linproj(ke.reshape(B, N, D), va.reshape(B, N, D),
                          lin_pk[i].T, lin_pv[i].T)
        x3 = _attn(q.reshape(B, N, D), k_, v_, xf.reshape(B, N, D),
                   lin_wo[i], _row(lin_bo[i]))
        xf = _ffn(x3.reshape(B * N, D), _row(lin_ln2_g[i]), _row(lin_ln2_b[i]),
                  lin_w1[i], _row(lin_b1[i]), lin_w2[i], _row(lin_b2[i]))

    for i in range(L):
        q, ke, va = _qkv(xf, _row(conv_ln1_g[i]), _row(conv_ln1_b[i]),
                         conv_wq[i], conv_wk[i], conv_wv[i])
        pkT = jnp.transpose(conv_pk[i], (2, 1, 0))
        pvT = jnp.transpose(conv_pv[i], (2, 1, 0))
        k_, v_ = _convproj(ke.reshape(B, K, S * D), va.reshape(B, K, S * D),
                           pkT, pvT)
        x3 = _attn(q.reshape(B, N, D), k_, v_, xf.reshape(B, N, D),
                   conv_wo[i], _row(conv_bo[i]))
        xf = _ffn(x3.reshape(B * N, D), _row(conv_ln2_g[i]), _row(conv_ln2_b[i]),
                  conv_w1[i], _row(conv_b1[i]), conv_w2[i], _row(conv_b2[i]))

    return xf.reshape(B, N, D)


# speedup vs baseline: 1.8194x; 1.0662x over previous
import jax
import jax.numpy as jnp
from jax.experimental import pallas as pl
from jax.experimental.pallas import tpu as pltpu

_INTERPRET = False
_H = 8


def _ln(x, g, b, eps=1e-5):
    m = jnp.mean(x, -1, keepdims=True)
    v = jnp.mean((x - m) ** 2, -1, keepdims=True)
    return (x - m) * jax.lax.rsqrt(v + eps) * g + b


def _pick_tile(total, want):
    t = min(want, total)
    while total % t:
        t //= 2
    return t


def _cparams(sem, vmem_mb=50):
    return pltpu.CompilerParams(
        dimension_semantics=sem, vmem_limit_bytes=vmem_mb * 1024 * 1024)


def _qkv_body(x_ref, g_ref, b_ref, wq_ref, wk_ref, wv_ref, q_ref, k_ref, v_ref):
    xn = _ln(x_ref[...], g_ref[...], b_ref[...])
    q_ref[...] = jnp.dot(xn, wq_ref[...], preferred_element_type=jnp.float32)
    k_ref[...] = jnp.dot(xn, wk_ref[...], preferred_element_type=jnp.float32)
    v_ref[...] = jnp.dot(xn, wv_ref[...], preferred_element_type=jnp.float32)


def _qkv(xf, g, b, wq, wk, wv):
    BN, D = xf.shape
    TN = _pick_tile(BN, 256)
    grid = (BN // TN,)
    row = pl.BlockSpec((TN, D), lambda i: (i, 0))
    full = pl.BlockSpec((D, D), lambda i: (0, 0))
    vec = pl.BlockSpec((1, D), lambda i: (0, 0))
    out = jax.ShapeDtypeStruct((BN, D), jnp.float32)
    return pl.pallas_call(
        _qkv_body,
        grid=grid,
        in_specs=[row, vec, vec, full, full, full],
        out_specs=[row, row, row],
        out_shape=[out, out, out],
        compiler_params=_cparams(("parallel",)),
        name="ln_qkv",
        interpret=_INTERPRET,
    )(xf, g, b, wq, wk, wv)


def _linproj_body(pk_ref, pv_ref, ke_ref, va_ref, ko_ref, vo_ref):
    j = pl.program_id(1)
    Bn = ke_ref.shape[0]
    for bb in range(Bn):
        kk = jax.lax.dot_general(pk_ref[...], ke_ref[bb], (((0,), (0,)), ((), ())),
                                 preferred_element_type=jnp.float32)
        vv = jax.lax.dot_general(pv_ref[...], va_ref[bb], (((0,), (0,)), ((), ())),
                                 preferred_element_type=jnp.float32)

        @pl.when(j == 0)
        def _(bb=bb, kk=kk, vv=vv):
            ko_ref[bb] = kk
            vo_ref[bb] = vv

        @pl.when(j != 0)
        def _(bb=bb, kk=kk, vv=vv):
            ko_ref[bb] += kk
            vo_ref[bb] += vv


def _linproj(ke, va, pk, pv):
    B, N, D = ke.shape
    K = pk.shape[1]
    BD = _pick_tile(D, D // 2)
    NC = _pick_tile(N, 1024)
    grid = (D // BD, N // NC)
    act = pl.BlockSpec((B, NC, BD), lambda d, j: (0, j, d))
    proj = pl.BlockSpec((NC, K), lambda d, j: (j, 0))
    out = pl.BlockSpec((B, K, BD), lambda d, j: (0, 0, d))
    osh = jax.ShapeDtypeStruct((B, K, D), jnp.float32)
    return pl.pallas_call(
        _linproj_body,
        grid=grid,
        in_specs=[proj, proj, act, act],
        out_specs=[out, out],
        out_shape=[osh, osh],
        compiler_params=_cparams(("parallel", "arbitrary")),
        name="lin_kv_proj",
        interpret=_INTERPRET,
    )(pk, pv, ke, va)


def _convproj_body(pk_ref, pv_ref, ke_ref, va_ref, ko_ref, vo_ref):
    s = pl.program_id(1)
    Bn = ke_ref.shape[0]
    dn = (((1,), (0,)), ((), ()))
    for bb in range(Bn):
        kk = jax.lax.dot_general(ke_ref[bb], pk_ref[0], dn,
                                 preferred_element_type=jnp.float32)
        vv = jax.lax.dot_general(va_ref[bb], pv_ref[0], dn,
                                 preferred_element_type=jnp.float32)

        @pl.when(s == 0)
        def _(bb=bb, kk=kk, vv=vv):
            ko_ref[bb] = kk
            vo_ref[bb] = vv

        @pl.when(s != 0)
        def _(bb=bb, kk=kk, vv=vv):
            ko_ref[bb] += kk
            vo_ref[bb] += vv


def _convproj(ke5, va5, pkT, pvT):
    B, K, SD = ke5.shape
    S, C, D = pkT.shape
    BO = _pick_tile(D, D // 2)
    grid = (D // BO, S)
    act = pl.BlockSpec((B, K, C), lambda o, s: (0, 0, s))
    w = pl.BlockSpec((1, C, BO), lambda o, s: (s, 0, o))
    out = pl.BlockSpec((B, K, BO), lambda o, s: (0, 0, o))
    osh = jax.ShapeDtypeStruct((B, K, D), jnp.float32)
    return pl.pallas_call(
        _convproj_body,
        grid=grid,
        in_specs=[w, w, act, act],
        out_specs=[out, out],
        out_shape=[osh, osh],
        compiler_params=_cparams(("parallel", "arbitrary")),
        name="conv_kv_proj",
        interpret=_INTERPRET,
    )(pkT, pvT, ke5, va5)


def _attn_ffn_body(q_ref, k_ref, v_ref, x_ref, wo_ref, bo_ref,
                   g_ref, b_ref, w1_ref, b1_ref, w2_ref, b2_ref, o_ref, *, nchunk):
    q = q_ref[0]
    k = k_ref[0]
    v = v_ref[0]
    D = q.shape[1]
    DH = D // _H
    scale = DH ** -0.5
    outs = []
    for h in range(_H):
        sl = slice(h * DH, (h + 1) * DH)
        dots = jax.lax.dot_general(q[:, sl], k[:, sl], (((1,), (1,)), ((), ())),
                                   preferred_element_type=jnp.float32) * scale
        m = jnp.max(dots, axis=-1, keepdims=True)
        p = jnp.exp(dots - m)
        l = jnp.sum(p, axis=-1, keepdims=True)
        a = p / l
        outs.append(jax.lax.dot_general(a, v[:, sl], (((1,), (0,)), ((), ())),
                                        preferred_element_type=jnp.float32))
    o = jnp.concatenate(outs, axis=-1)
    x1 = x_ref[0] + jnp.dot(o, wo_ref[...], preferred_element_type=jnp.float32) + bo_ref[...]
    xn = _ln(x1, g_ref[...], b_ref[...])
    DFF = w1_ref.shape[1]
    CF = DFF // nchunk
    o_ref[0] = x1 + b2_ref[...]
    for c in range(nchunk):
        slc = slice(c * CF, (c + 1) * CF)
        h = jnp.dot(xn, w1_ref[:, slc], preferred_element_type=jnp.float32) + b1_ref[:, slc]
        h = 0.5 * h * (1.0 + jax.lax.erf(h * (2.0 ** -0.5)))
        o_ref[0] += jnp.dot(h, w2_ref[slc, :], preferred_element_type=jnp.float32)


def _attn_ffn(q3, k_, v_, x3, wo, bo, g, b, w1, b1, w2, b2):
    import functools
    B, N, D = q3.shape
    K = k_.shape[1]
    DFF = w1.shape[1]
    TQ = _pick_tile(N, 256)
    CF = _pick_tile(DFF, 512)
    grid = (B, N // TQ)
    row = pl.BlockSpec((1, TQ, D), lambda bb, n: (bb, n, 0))
    kv = pl.BlockSpec((1, K, D), lambda bb, n: (bb, 0, 0))
    full = pl.BlockSpec((D, D), lambda bb, n: (0, 0))
    vec = pl.BlockSpec((1, D), lambda bb, n: (0, 0))
    return pl.pallas_call(
        functools.partial(_attn_ffn_body, nchunk=DFF // CF),
        grid=grid,
        in_specs=[row, kv, kv, row, full, vec,
                  vec, vec,
                  pl.BlockSpec((D, DFF), lambda bb, n: (0, 0)),
                  pl.BlockSpec((1, DFF), lambda bb, n: (0, 0)),
                  pl.BlockSpec((DFF, D), lambda bb, n: (0, 0)),
                  vec],
        out_specs=row,
        out_shape=jax.ShapeDtypeStruct((B, N, D), jnp.float32),
        compiler_params=_cparams(("parallel", "parallel")),
        name="attn_ffn",
        interpret=_INTERPRET,
    )(q3, k_, v_, x3, wo, bo, g, b, w1, b1, w2, b2)


def _row(v):
    return v.reshape(1, -1)


def kernel(x, lin_ln1_g, lin_ln1_b, lin_wq, lin_wk, lin_wv, lin_pk, lin_pv,
           lin_wo, lin_bo, lin_ln2_g, lin_ln2_b, lin_w1, lin_b1, lin_w2, lin_b2,
           conv_ln1_g, conv_ln1_b, conv_wq, conv_wk, conv_wv, conv_pk, conv_pv,
           conv_wo, conv_bo, conv_ln2_g, conv_ln2_b, conv_w1, conv_b1, conv_w2, conv_b2):
    B, N, D = x.shape
    L = lin_wq.shape[0]
    K = lin_pk.shape[2]
    S = conv_pk.shape[3]
    xf = x.reshape(B * N, D)

    for i in range(L):
        q, ke, va = _qkv(xf, _row(lin_ln1_g[i]), _row(lin_ln1_b[i]),
                         lin_wq[i], lin_wk[i], lin_wv[i])
        k_, v_ = _linproj(ke.reshape(B, N, D), va.reshape(B, N, D),
                          lin_pk[i], lin_pv[i])
        x3 = _attn_ffn(q.reshape(B, N, D), k_, v_, xf.reshape(B, N, D),
                       lin_wo[i], _row(lin_bo[i]),
                       _row(lin_ln2_g[i]), _row(lin_ln2_b[i]),
                       lin_w1[i], _row(lin_b1[i]), lin_w2[i], _row(lin_b2[i]))
        xf = x3.reshape(B * N, D)

    pkT_all = jnp.transpose(conv_pk, (0, 3, 2, 1))
    pvT_all = jnp.transpose(conv_pv, (0, 3, 2, 1))
    for i in range(L):
        q, ke, va = _qkv(xf, _row(conv_ln1_g[i]), _row(conv_ln1_b[i]),
                         conv_wq[i], conv_wk[i], conv_wv[i])
        k_, v_ = _convproj(ke.reshape(B, K, S * D), va.reshape(B, K, S * D),
                           pkT_all[i], pvT_all[i])
        x3 = _attn_ffn(q.reshape(B, N, D), k_, v_, xf.reshape(B, N, D),
                       conv_wo[i], _row(conv_bo[i]),
                       _row(conv_ln2_g[i]), _row(conv_ln2_b[i]),
                       conv_w1[i], _row(conv_b1[i]), conv_w2[i], _row(conv_b2[i]))
        xf = x3.reshape(B * N, D)

    return xf.reshape(B, N, D)


# speedup vs baseline: 2.1785x; 1.1974x over previous
import jax
import jax.numpy as jnp
from jax.experimental import pallas as pl
from jax.experimental.pallas import tpu as pltpu

_INTERPRET = False
_H = 8


def _ln(x, g, b, eps=1e-5):
    m = jnp.mean(x, -1, keepdims=True)
    v = jnp.mean((x - m) ** 2, -1, keepdims=True)
    return (x - m) * jax.lax.rsqrt(v + eps) * g + b


def _pick_tile(total, want):
    t = min(want, total)
    while total % t:
        t //= 2
    return t


def _cparams(sem, vmem_mb=50):
    return pltpu.CompilerParams(
        dimension_semantics=sem, vmem_limit_bytes=vmem_mb * 1024 * 1024)


def _qkv_body(x_ref, g_ref, b_ref, wq_ref, wk_ref, wv_ref, q_ref, k_ref, v_ref):
    xn = _ln(x_ref[...], g_ref[...], b_ref[...])
    q_ref[...] = jnp.dot(xn, wq_ref[...], preferred_element_type=jnp.float32)
    k_ref[...] = jnp.dot(xn, wk_ref[...], preferred_element_type=jnp.float32)
    v_ref[...] = jnp.dot(xn, wv_ref[...], preferred_element_type=jnp.float32)


def _qkv(xf, g, b, wq, wk, wv):
    BN, D = xf.shape
    TN = _pick_tile(BN, 256)
    grid = (BN // TN,)
    row = pl.BlockSpec((TN, D), lambda i: (i, 0))
    full = pl.BlockSpec((D, D), lambda i: (0, 0))
    vec = pl.BlockSpec((1, D), lambda i: (0, 0))
    out = jax.ShapeDtypeStruct((BN, D), jnp.float32)
    return pl.pallas_call(
        _qkv_body,
        grid=grid,
        in_specs=[row, vec, vec, full, full, full],
        out_specs=[row, row, row],
        out_shape=[out, out, out],
        compiler_params=_cparams(("parallel",)),
        name="ln_qkv",
        interpret=_INTERPRET,
    )(xf, g, b, wq, wk, wv)


def _linproj_body(pk_ref, pv_ref, ke_ref, va_ref, ko_ref, vo_ref):
    j = pl.program_id(1)
    Bn = ke_ref.shape[0]
    for bb in range(Bn):
        kk = jax.lax.dot_general(pk_ref[...], ke_ref[bb], (((0,), (0,)), ((), ())),
                                 preferred_element_type=jnp.float32)
        vv = jax.lax.dot_general(pv_ref[...], va_ref[bb], (((0,), (0,)), ((), ())),
                                 preferred_element_type=jnp.float32)

        @pl.when(j == 0)
        def _(bb=bb, kk=kk, vv=vv):
            ko_ref[bb] = kk
            vo_ref[bb] = vv

        @pl.when(j != 0)
        def _(bb=bb, kk=kk, vv=vv):
            ko_ref[bb] += kk
            vo_ref[bb] += vv


def _linproj(ke, va, pk, pv):
    B, N, D = ke.shape
    K = pk.shape[1]
    BD = _pick_tile(D, D // 2)
    NC = _pick_tile(N, 1024)
    grid = (D // BD, N // NC)
    act = pl.BlockSpec((B, NC, BD), lambda d, j: (0, j, d))
    proj = pl.BlockSpec((NC, K), lambda d, j: (j, 0))
    out = pl.BlockSpec((B, K, BD), lambda d, j: (0, 0, d))
    osh = jax.ShapeDtypeStruct((B, K, D), jnp.float32)
    return pl.pallas_call(
        _linproj_body,
        grid=grid,
        in_specs=[proj, proj, act, act],
        out_specs=[out, out],
        out_shape=[osh, osh],
        compiler_params=_cparams(("parallel", "arbitrary")),
        name="lin_kv_proj",
        interpret=_INTERPRET,
    )(pk, pv, ke, va)


def _convproj_body(wk_hbm, wv_hbm, ke_hbm, va_hbm, ko_ref, vo_ref,
                   wkb, wvb, keb, vab, sem):
    S = wk_hbm.shape[1]
    Bn = ke_hbm.shape[0]
    s = pl.program_id(0)

    def start(sidx, slot):
        pltpu.make_async_copy(wk_hbm.at[:, pl.ds(sidx, 1), :], wkb.at[slot],
                              sem.at[slot, 0]).start()
        pltpu.make_async_copy(wv_hbm.at[:, pl.ds(sidx, 1), :], wvb.at[slot],
                              sem.at[slot, 1]).start()
        pltpu.make_async_copy(ke_hbm.at[:, :, pl.ds(sidx, 1), :], keb.at[slot],
                              sem.at[slot, 2]).start()
        pltpu.make_async_copy(va_hbm.at[:, :, pl.ds(sidx, 1), :], vab.at[slot],
                              sem.at[slot, 3]).start()

    slot = jax.lax.rem(s, 2)

    @pl.when(s == 0)
    def _():
        start(0, 0)

    @pl.when(s < S - 1)
    def _():
        start(s + 1, 1 - slot)

    pltpu.make_async_copy(wkb.at[slot], wkb.at[slot], sem.at[slot, 0]).wait()
    pltpu.make_async_copy(wvb.at[slot], wvb.at[slot], sem.at[slot, 1]).wait()
    pltpu.make_async_copy(keb.at[slot], keb.at[slot], sem.at[slot, 2]).wait()
    pltpu.make_async_copy(vab.at[slot], vab.at[slot], sem.at[slot, 3]).wait()

    dn = (((1,), (1,)), ((), ()))
    wk = wkb[slot, :, 0, :]
    wv = wvb[slot, :, 0, :]
    for bb in range(Bn):
        kk = jax.lax.dot_general(keb[slot, bb, :, 0, :], wk, dn,
                                 preferred_element_type=jnp.float32)
        vv = jax.lax.dot_general(vab[slot, bb, :, 0, :], wv, dn,
                                 preferred_element_type=jnp.float32)

        @pl.when(s == 0)
        def _(bb=bb, kk=kk, vv=vv):
            ko_ref[bb] = kk
            vo_ref[bb] = vv

        @pl.when(s != 0)
        def _(bb=bb, kk=kk, vv=vv):
            ko_ref[bb] += kk
            vo_ref[bb] += vv


def _convproj(ke4, va4, wkp, wvp):
    B, K, S, D = ke4.shape
    O = wkp.shape[0]
    osh = jax.ShapeDtypeStruct((B, K, D), jnp.float32)
    out = pl.BlockSpec((B, K, D), lambda s: (0, 0, 0))
    anyspec = pl.BlockSpec(memory_space=pl.ANY)
    return pl.pallas_call(
        _convproj_body,
        grid=(S,),
        in_specs=[anyspec, anyspec, anyspec, anyspec],
        out_specs=[out, out],
        out_shape=[osh, osh],
        scratch_shapes=[
            pltpu.VMEM((2, O, 1, D), jnp.float32),
            pltpu.VMEM((2, O, 1, D), jnp.float32),
            pltpu.VMEM((2, B, K, 1, D), jnp.float32),
            pltpu.VMEM((2, B, K, 1, D), jnp.float32),
            pltpu.SemaphoreType.DMA((2, 4)),
        ],
        compiler_params=_cparams(("arbitrary",)),
        name="conv_kv_proj",
        interpret=_INTERPRET,
    )(wkp, wvp, ke4, va4)


def _attn_ffn_body(q_ref, k_ref, v_ref, x_ref, wo_ref, bo_ref,
                   g_ref, b_ref, w1_ref, b1_ref, w2_ref, b2_ref, o_ref, *, nchunk):
    q = q_ref[0]
    k = k_ref[0]
    v = v_ref[0]
    D = q.shape[1]
    DH = D // _H
    scale = DH ** -0.5
    outs = []
    for h in range(_H):
        sl = slice(h * DH, (h + 1) * DH)
        dots = jax.lax.dot_general(q[:, sl], k[:, sl], (((1,), (1,)), ((), ())),
                                   preferred_element_type=jnp.float32) * scale
        m = jnp.max(dots, axis=-1, keepdims=True)
        p = jnp.exp(dots - m)
        l = jnp.sum(p, axis=-1, keepdims=True)
        a = p / l
        outs.append(jax.lax.dot_general(a, v[:, sl], (((1,), (0,)), ((), ())),
                                        preferred_element_type=jnp.float32))
    o = jnp.concatenate(outs, axis=-1)
    x1 = x_ref[0] + jnp.dot(o, wo_ref[...], preferred_element_type=jnp.float32) + bo_ref[...]
    xn = _ln(x1, g_ref[...], b_ref[...])
    DFF = w1_ref.shape[1]
    CF = DFF // nchunk
    o_ref[0] = x1 + b2_ref[...]
    for c in range(nchunk):
        slc = slice(c * CF, (c + 1) * CF)
        h = jnp.dot(xn, w1_ref[:, slc], preferred_element_type=jnp.float32) + b1_ref[:, slc]
        h = 0.5 * h * (1.0 + jax.lax.erf(h * (2.0 ** -0.5)))
        o_ref[0] += jnp.dot(h, w2_ref[slc, :], preferred_element_type=jnp.float32)


def _attn_ffn(q3, k_, v_, x3, wo, bo, g, b, w1, b1, w2, b2):
    import functools
    B, N, D = q3.shape
    K = k_.shape[1]
    DFF = w1.shape[1]
    TQ = _pick_tile(N, 256)
    CF = _pick_tile(DFF, 512)
    grid = (B, N // TQ)
    row = pl.BlockSpec((1, TQ, D), lambda bb, n: (bb, n, 0))
    kv = pl.BlockSpec((1, K, D), lambda bb, n: (bb, 0, 0))
    full = pl.BlockSpec((D, D), lambda bb, n: (0, 0))
    vec = pl.BlockSpec((1, D), lambda bb, n: (0, 0))
    return pl.pallas_call(
        functools.partial(_attn_ffn_body, nchunk=DFF // CF),
        grid=grid,
        in_specs=[row, kv, kv, row, full, vec,
                  vec, vec,
                  pl.BlockSpec((D, DFF), lambda bb, n: (0, 0)),
                  pl.BlockSpec((1, DFF), lambda bb, n: (0, 0)),
                  pl.BlockSpec((DFF, D), lambda bb, n: (0, 0)),
                  vec],
        out_specs=row,
        out_shape=jax.ShapeDtypeStruct((B, N, D), jnp.float32),
        compiler_params=_cparams(("parallel", "parallel")),
        name="attn_ffn",
        interpret=_INTERPRET,
    )(q3, k_, v_, x3, wo, bo, g, b, w1, b1, w2, b2)


def _row(v):
    return v.reshape(1, -1)


def kernel(x, lin_ln1_g, lin_ln1_b, lin_wq, lin_wk, lin_wv, lin_pk, lin_pv,
           lin_wo, lin_bo, lin_ln2_g, lin_ln2_b, lin_w1, lin_b1, lin_w2, lin_b2,
           conv_ln1_g, conv_ln1_b, conv_wq, conv_wk, conv_wv, conv_pk, conv_pv,
           conv_wo, conv_bo, conv_ln2_g, conv_ln2_b, conv_w1, conv_b1, conv_w2, conv_b2):
    B, N, D = x.shape
    L = lin_wq.shape[0]
    K = lin_pk.shape[2]
    S = conv_pk.shape[3]
    xf = x.reshape(B * N, D)

    for i in range(L):
        q, ke, va = _qkv(xf, _row(lin_ln1_g[i]), _row(lin_ln1_b[i]),
                         lin_wq[i], lin_wk[i], lin_wv[i])
        k_, v_ = _linproj(ke.reshape(B, N, D), va.reshape(B, N, D),
                          lin_pk[i], lin_pv[i])
        x3 = _attn_ffn(q.reshape(B, N, D), k_, v_, xf.reshape(B, N, D),
                       lin_wo[i], _row(lin_bo[i]),
                       _row(lin_ln2_g[i]), _row(lin_ln2_b[i]),
                       lin_w1[i], _row(lin_b1[i]), lin_w2[i], _row(lin_b2[i]))
        xf = x3.reshape(B * N, D)

    pkT_all = jnp.transpose(conv_pk, (0, 1, 3, 2))
    pvT_all = jnp.transpose(conv_pv, (0, 1, 3, 2))
    for i in range(L):
        q, ke, va = _qkv(xf, _row(conv_ln1_g[i]), _row(conv_ln1_b[i]),
                         conv_wq[i], conv_wk[i], conv_wv[i])
        k_, v_ = _convproj(ke.reshape(B, K, S, D), va.reshape(B, K, S, D),
                           pkT_all[i], pvT_all[i])
        x3 = _attn_ffn(q.reshape(B, N, D), k_, v_, xf.reshape(B, N, D),
                       conv_wo[i], _row(conv_bo[i]),
                       _row(conv_ln2_g[i]), _row(conv_ln2_b[i]),
                       conv_w1[i], _row(conv_b1[i]), conv_w2[i], _row(conv_b2[i]))
        xf = x3.reshape(B * N, D)

    return xf.reshape(B, N, D)


# speedup vs baseline: 2.6272x; 1.2060x over previous
import functools

import jax
import jax.numpy as jnp
from jax.experimental import pallas as pl
from jax.experimental.pallas import tpu as pltpu

_INTERPRET = False
_H = 8


def _ln(x, g, b, eps=1e-5):
    m = jnp.mean(x, -1, keepdims=True)
    v = jnp.mean((x - m) ** 2, -1, keepdims=True)
    return (x - m) * jax.lax.rsqrt(v + eps) * g + b


def _pick_tile(total, want):
    t = min(want, total)
    while total % t:
        t //= 2
    return t


def _cparams(sem, vmem_mb=50):
    return pltpu.CompilerParams(
        dimension_semantics=sem, vmem_limit_bytes=vmem_mb * 1024 * 1024)


def _qkv_body(x_ref, g_ref, b_ref, wq_ref, wk_ref, wv_ref, q_ref, k_ref, v_ref, *, li):
    xn = _ln(x_ref[...], g_ref[li:li + 1, :], b_ref[li:li + 1, :])
    q_ref[...] = jnp.dot(xn, wq_ref[0], preferred_element_type=jnp.float32)
    k_ref[...] = jnp.dot(xn, wk_ref[0], preferred_element_type=jnp.float32)
    v_ref[...] = jnp.dot(xn, wv_ref[0], preferred_element_type=jnp.float32)


def _qkv(xf, li, g, b, wq, wk, wv):
    BN, D = xf.shape
    TN = _pick_tile(BN, 256)
    grid = (BN // TN,)
    L = g.shape[0]
    row = pl.BlockSpec((TN, D), lambda i: (i, 0))
    full = pl.BlockSpec((1, D, D), lambda i: (li, 0, 0))
    vec = pl.BlockSpec((L, D), lambda i: (0, 0))
    out = jax.ShapeDtypeStruct((BN, D), jnp.float32)
    return pl.pallas_call(
        functools.partial(_qkv_body, li=li),
        grid=grid,
        in_specs=[row, vec, vec, full, full, full],
        out_specs=[row, row, row],
        out_shape=[out, out, out],
        compiler_params=_cparams(("parallel",)),
        name="ln_qkv",
        interpret=_INTERPRET,
    )(xf, g, b, wq, wk, wv)


def _linproj_body(pk_ref, pv_ref, ke_ref, va_ref, ko_ref, vo_ref):
    j = pl.program_id(1)
    Bn = ke_ref.shape[0]
    for bb in range(Bn):
        kk = jax.lax.dot_general(pk_ref[0], ke_ref[bb], (((0,), (0,)), ((), ())),
                                 preferred_element_type=jnp.float32)
        vv = jax.lax.dot_general(pv_ref[0], va_ref[bb], (((0,), (0,)), ((), ())),
                                 preferred_element_type=jnp.float32)

        @pl.when(j == 0)
        def _(bb=bb, kk=kk, vv=vv):
            ko_ref[bb] = kk
            vo_ref[bb] = vv

        @pl.when(j != 0)
        def _(bb=bb, kk=kk, vv=vv):
            ko_ref[bb] += kk
            vo_ref[bb] += vv


def _linproj(ke, va, li, pk, pv):
    B, N, D = ke.shape
    K = pk.shape[2]
    BD = _pick_tile(D, D // 2)
    NC = _pick_tile(N, 1024)
    grid = (D // BD, N // NC)
    act = pl.BlockSpec((B, NC, BD), lambda d, j: (0, j, d))
    proj = pl.BlockSpec((1, NC, K), lambda d, j: (li, j, 0))
    out = pl.BlockSpec((B, K, BD), lambda d, j: (0, 0, d))
    osh = jax.ShapeDtypeStruct((B, K, D), jnp.float32)
    return pl.pallas_call(
        _linproj_body,
        grid=grid,
        in_specs=[proj, proj, act, act],
        out_specs=[out, out],
        out_shape=[osh, osh],
        compiler_params=_cparams(("parallel", "arbitrary")),
        name="lin_kv_proj",
        interpret=_INTERPRET,
    )(pk, pv, ke, va)


def _convproj_body(wk_hbm, wv_hbm, ke_hbm, va_hbm, ko_ref, vo_ref,
                   wkb, wvb, keb, vab, sem, *, li):
    S = wk_hbm.shape[2]
    Bn = ke_hbm.shape[0]
    s = pl.program_id(0)

    def start(sidx, slot):
        pltpu.make_async_copy(wk_hbm.at[li, :, pl.ds(sidx, 1), :], wkb.at[slot],
                              sem.at[slot, 0]).start()
        pltpu.make_async_copy(wv_hbm.at[li, :, pl.ds(sidx, 1), :], wvb.at[slot],
                              sem.at[slot, 1]).start()
        pltpu.make_async_copy(ke_hbm.at[:, :, pl.ds(sidx, 1), :], keb.at[slot],
                              sem.at[slot, 2]).start()
        pltpu.make_async_copy(va_hbm.at[:, :, pl.ds(sidx, 1), :], vab.at[slot],
                              sem.at[slot, 3]).start()

    slot = jax.lax.rem(s, 2)

    @pl.when(s == 0)
    def _():
        start(0, 0)

    @pl.when(s < S - 1)
    def _():
        start(s + 1, 1 - slot)

    pltpu.make_async_copy(wkb.at[slot], wkb.at[slot], sem.at[slot, 0]).wait()
    pltpu.make_async_copy(wvb.at[slot], wvb.at[slot], sem.at[slot, 1]).wait()
    pltpu.make_async_copy(keb.at[slot], keb.at[slot], sem.at[slot, 2]).wait()
    pltpu.make_async_copy(vab.at[slot], vab.at[slot], sem.at[slot, 3]).wait()

    dn = (((1,), (1,)), ((), ()))
    wk = wkb[slot, :, 0, :]
    wv = wvb[slot, :, 0, :]
    for bb in range(Bn):
        kk = jax.lax.dot_general(keb[slot, bb, :, 0, :], wk, dn,
                                 preferred_element_type=jnp.float32)
        vv = jax.lax.dot_general(vab[slot, bb, :, 0, :], wv, dn,
                                 preferred_element_type=jnp.float32)

        @pl.when(s == 0)
        def _(bb=bb, kk=kk, vv=vv):
            ko_ref[bb] = kk
            vo_ref[bb] = vv

        @pl.when(s != 0)
        def _(bb=bb, kk=kk, vv=vv):
            ko_ref[bb] += kk
            vo_ref[bb] += vv


def _convproj(ke4, va4, li, wkp, wvp):
    B, K, S, D = ke4.shape
    O = wkp.shape[1]
    osh = jax.ShapeDtypeStruct((B, K, D), jnp.float32)
    out = pl.BlockSpec((B, K, D), lambda s: (0, 0, 0))
    anyspec = pl.BlockSpec(memory_space=pl.ANY)
    return pl.pallas_call(
        functools.partial(_convproj_body, li=li),
        grid=(S,),
        in_specs=[anyspec, anyspec, anyspec, anyspec],
        out_specs=[out, out],
        out_shape=[osh, osh],
        scratch_shapes=[
            pltpu.VMEM((2, O, 1, D), jnp.float32),
            pltpu.VMEM((2, O, 1, D), jnp.float32),
            pltpu.VMEM((2, B, K, 1, D), jnp.float32),
            pltpu.VMEM((2, B, K, 1, D), jnp.float32),
            pltpu.SemaphoreType.DMA((2, 4)),
        ],
        compiler_params=_cparams(("arbitrary",)),
        name="conv_kv_proj",
        interpret=_INTERPRET,
    )(wkp, wvp, ke4, va4)


def _attn_ffn_body(q_ref, k_ref, v_ref, x_ref, wo_ref, bo_ref,
                   g_ref, b_ref, w1_ref, b1_ref, w2_ref, b2_ref, o_ref, *, nchunk, li):
    q = q_ref[0]
    k = k_ref[0]
    v = v_ref[0]
    D = q.shape[1]
    DH = D // _H
    scale = DH ** -0.5
    outs = []
    for h in range(_H):
        sl = slice(h * DH, (h + 1) * DH)
        dots = jax.lax.dot_general(q[:, sl], k[:, sl], (((1,), (1,)), ((), ())),
                                   preferred_element_type=jnp.float32) * scale
        m = jnp.max(dots, axis=-1, keepdims=True)
        p = jnp.exp(dots - m)
        l = jnp.sum(p, axis=-1, keepdims=True)
        a = p / l
        outs.append(jax.lax.dot_general(a, v[:, sl], (((1,), (0,)), ((), ())),
                                        preferred_element_type=jnp.float32))
    o = jnp.concatenate(outs, axis=-1)
    x1 = x_ref[0] + jnp.dot(o, wo_ref[0], preferred_element_type=jnp.float32) + bo_ref[li:li + 1, :]
    xn = _ln(x1, g_ref[li:li + 1, :], b_ref[li:li + 1, :])
    DFF = w1_ref.shape[2]
    CF = DFF // nchunk
    o_ref[0] = x1 + b2_ref[li:li + 1, :]
    for c in range(nchunk):
        slc = slice(c * CF, (c + 1) * CF)
        h = jnp.dot(xn, w1_ref[0, :, slc], preferred_element_type=jnp.float32) + b1_ref[li:li + 1, slc]
        h = 0.5 * h * (1.0 + jax.lax.erf(h * (2.0 ** -0.5)))
        o_ref[0] += jnp.dot(h, w2_ref[0, slc, :], preferred_element_type=jnp.float32)


def _attn_ffn(q3, k_, v_, x3, li, wo, bo, g, b, w1, b1, w2, b2):
    B, N, D = q3.shape
    K = k_.shape[1]
    DFF = w1.shape[2]
    TQ = _pick_tile(N, 256)
    CF = _pick_tile(DFF, 512)
    L = bo.shape[0]
    grid = (B, N // TQ)
    row = pl.BlockSpec((1, TQ, D), lambda bb, n: (bb, n, 0))
    kv = pl.BlockSpec((1, K, D), lambda bb, n: (bb, 0, 0))
    full = pl.BlockSpec((1, D, D), lambda bb, n: (li, 0, 0))
    vec = pl.BlockSpec((L, D), lambda bb, n: (0, 0))
    return pl.pallas_call(
        functools.partial(_attn_ffn_body, nchunk=DFF // CF, li=li),
        grid=grid,
        in_specs=[row, kv, kv, row, full, vec,
                  vec, vec,
                  pl.BlockSpec((1, D, DFF), lambda bb, n: (li, 0, 0)),
                  pl.BlockSpec((L, DFF), lambda bb, n: (0, 0)),
                  pl.BlockSpec((1, DFF, D), lambda bb, n: (li, 0, 0)),
                  vec],
        out_specs=row,
        out_shape=jax.ShapeDtypeStruct((B, N, D), jnp.float32),
        compiler_params=_cparams(("parallel", "parallel")),
        name="attn_ffn",
        interpret=_INTERPRET,
    )(q3, k_, v_, x3, wo, bo, g, b, w1, b1, w2, b2)


def kernel(x, lin_ln1_g, lin_ln1_b, lin_wq, lin_wk, lin_wv, lin_pk, lin_pv,
           lin_wo, lin_bo, lin_ln2_g, lin_ln2_b, lin_w1, lin_b1, lin_w2, lin_b2,
           conv_ln1_g, conv_ln1_b, conv_wq, conv_wk, conv_wv, conv_pk, conv_pv,
           conv_wo, conv_bo, conv_ln2_g, conv_ln2_b, conv_w1, conv_b1, conv_w2, conv_b2):
    B, N, D = x.shape
    L = lin_wq.shape[0]
    K = lin_pk.shape[2]
    S = conv_pk.shape[3]
    xf = x.reshape(B * N, D)

    for i in range(L):
        q, ke, va = _qkv(xf, i, lin_ln1_g, lin_ln1_b, lin_wq, lin_wk, lin_wv)
        k_, v_ = _linproj(ke.reshape(B, N, D), va.reshape(B, N, D), i,
                          lin_pk, lin_pv)
        x3 = _attn_ffn(q.reshape(B, N, D), k_, v_, xf.reshape(B, N, D), i,
                       lin_wo, lin_bo, lin_ln2_g, lin_ln2_b,
                       lin_w1, lin_b1, lin_w2, lin_b2)
        xf = x3.reshape(B * N, D)

    pkT_all = jnp.transpose(conv_pk, (0, 1, 3, 2))
    pvT_all = jnp.transpose(conv_pv, (0, 1, 3, 2))
    for i in range(L):
        q, ke, va = _qkv(xf, i, conv_ln1_g, conv_ln1_b, conv_wq, conv_wk, conv_wv)
        k_, v_ = _convproj(ke.reshape(B, K, S, D), va.reshape(B, K, S, D), i,
                           pkT_all, pvT_all)
        x3 = _attn_ffn(q.reshape(B, N, D), k_, v_, xf.reshape(B, N, D), i,
                       conv_wo, conv_bo, conv_ln2_g, conv_ln2_b,
                       conv_w1, conv_b1, conv_w2, conv_b2)
        xf = x3.reshape(B * N, D)

    return xf.reshape(B, N, D)
